# Initial kernel scaffold; baseline (speedup 1.0000x reference)
import jax
import jax.numpy as jnp
from jax.experimental import pallas as pl


def kernel(inputs, emb):
    raise NotImplementedError("write your pallas kernel here")


# speedup vs baseline: 3.0128x; 3.0128x over previous
import functools

import jax
import jax.numpy as jnp
from jax import lax
from jax.experimental import pallas as pl
from jax.experimental.pallas import tpu as pltpu
from jax.experimental.pallas import tpu_sc as plsc


def kernel(inputs, emb):
    B, H = inputs.shape
    V, D = emb.shape
    NC, NS = 2, 16
    NW = NC * NS
    BPW = B // NW

    idx_prep = inputs.reshape(NW, BPW, H).transpose(0, 2, 1)

    mesh = plsc.VectorSubcoreMesh(
        core_axis_name="c", subcore_axis_name="s", num_cores=NC, num_subcores=NS
    )

    @functools.partial(
        pl.kernel,
        out_type=jax.ShapeDtypeStruct((B, D), jnp.float32),
        mesh=mesh,
        scratch_types=[
            pltpu.VMEM((H, BPW), jnp.int32),
            pltpu.VMEM((BPW, D), jnp.float32),
            pltpu.SemaphoreType.DMA,
        ],
        compiler_params=pltpu.CompilerParams(use_tc_tiling_on_sc=False),
    )
    def body(idx_hbm, emb_hbm, out_hbm, idx_v, acc_v, sem):
        wid = lax.axis_index("s") * NC + lax.axis_index("c")
        pltpu.sync_copy(idx_hbm.at[wid], idx_v)

        def zero_row(i, carry):
            z = jnp.zeros((16,), jnp.float32)
            acc_v[i, pl.ds(0, 16)] = z
            acc_v[i, pl.ds(16, 16)] = z
            return carry

        lax.fori_loop(0, BPW, zero_row, 0)

        def fire(j, carry):
            pltpu.async_copy(emb_hbm.at[idx_v.at[j]], acc_v, sem, add=True)
            return carry

        lax.fori_loop(0, H, fire, 0)

        def drain(j, carry):
            pltpu.make_async_copy(emb_hbm.at[idx_v.at[j]], acc_v, sem).wait()
            return carry

        lax.fori_loop(0, H, drain, 0)

        pltpu.sync_copy(acc_v, out_hbm.at[pl.ds(wid * BPW, BPW)])

    return body(idx_prep, emb)
